# Initial kernel scaffold; baseline (speedup 1.0000x reference)
import jax
import jax.numpy as jnp
from jax.experimental import pallas as pl


def kernel(features, edge_index, edge_values, W):
    raise NotImplementedError("write your pallas kernel here")


# speedup vs baseline: 4.1255x; 4.1255x over previous
import functools

import jax
import jax.numpy as jnp
from jax import lax
from jax.experimental import pallas as pl
from jax.experimental.pallas import tpu as pltpu
from jax.experimental.pallas import tpu_sc as plsc

N_NODES = 10000
N_EDGES = 320000
D = 128
LANES = 16

NUM_CORES = 2
NUM_SUBCORES = 16
NUM_WORKERS = NUM_CORES * NUM_SUBCORES
EDGES_PER_WORKER = N_EDGES // NUM_WORKERS
BATCH = 80
NUM_BATCHES = EDGES_PER_WORKER // BATCH
N_PAD = 10240
ROWS_PER_TILE = N_PAD // NUM_SUBCORES
ZERO_CHUNK = 80


def _sc_segment_sum(features, src, dst, val):
  mesh = plsc.VectorSubcoreMesh(core_axis_name="c", subcore_axis_name="s")

  @functools.partial(
      pl.kernel,
      mesh=mesh,
      out_type=jax.ShapeDtypeStruct((NUM_CORES, N_PAD, D), jnp.float32),
      scratch_types=[
          pltpu.VMEM((BATCH,), jnp.int32),
          pltpu.VMEM((BATCH,), jnp.int32),
          pltpu.VMEM((BATCH,), jnp.float32),
          pltpu.VMEM((BATCH, D), jnp.float32),
          pltpu.VMEM_SHARED((N_PAD, D), jnp.float32),
          pltpu.SemaphoreType.DMA,
      ],
  )
  def k(feat_hbm, src_hbm, dst_hbm, val_hbm, out_hbm,
        src_v, dst_v, val_v, rows_v, accum, sem):
    c = lax.axis_index("c")
    s = lax.axis_index("s")
    wid = s * NUM_CORES + c

    zeros = jnp.zeros((LANES,), jnp.float32)

    def zero_body(i, _):
      for j in range(D // LANES):
        rows_v[i, pl.ds(j * LANES, LANES)] = zeros
      return _

    lax.fori_loop(0, ZERO_CHUNK, zero_body, None)
    for kk in range(ROWS_PER_TILE // ZERO_CHUNK):
      pltpu.sync_copy(
          rows_v.at[pl.ds(0, ZERO_CHUNK)],
          accum.at[pl.ds(s * ROWS_PER_TILE + kk * ZERO_CHUNK, ZERO_CHUNK)])
    plsc.subcore_barrier()

    def batch_body(t, _):
      off = wid * EDGES_PER_WORKER + t * BATCH
      pltpu.sync_copy(src_hbm.at[pl.ds(off, BATCH)], src_v)
      pltpu.sync_copy(dst_hbm.at[pl.ds(off, BATCH)], dst_v)
      pltpu.sync_copy(val_hbm.at[pl.ds(off, BATCH)], val_v)
      pltpu.async_copy(feat_hbm.at[src_v], rows_v, sem).wait()

      def scale_group(g, _):
        vv = val_v[pl.ds(g * LANES, LANES)]
        for lane in range(LANES):
          v = vv[lane]
          e = g * LANES + lane
          for j in range(D // LANES):
            sl = pl.ds(j * LANES, LANES)
            rows_v[e, sl] = rows_v[e, sl] * v
        return _

      lax.fori_loop(0, BATCH // LANES, scale_group, None)
      pltpu.sync_copy(rows_v, accum.at[dst_v], add=True)
      return _

    lax.fori_loop(0, NUM_BATCHES, batch_body, None)
    plsc.subcore_barrier()

    base = s * ROWS_PER_TILE
    pltpu.sync_copy(accum.at[pl.ds(base, ROWS_PER_TILE)],
                    out_hbm.at[c, pl.ds(base, ROWS_PER_TILE)])

  return k(features, src, dst, val)


def _tc_combine_matmul(p0, p1, w):
  block_rows = 1000

  def body(p0_ref, p1_ref, w_ref, out_ref):
    out_ref[...] = jnp.dot(p0_ref[...] + p1_ref[...], w_ref[...],
                           preferred_element_type=jnp.float32)

  return pl.pallas_call(
      body,
      grid=(N_NODES // block_rows,),
      in_specs=[
          pl.BlockSpec((block_rows, D), lambda i: (i, 0)),
          pl.BlockSpec((block_rows, D), lambda i: (i, 0)),
          pl.BlockSpec((D, D), lambda i: (0, 0)),
      ],
      out_specs=pl.BlockSpec((block_rows, D), lambda i: (i, 0)),
      out_shape=jax.ShapeDtypeStruct((N_NODES, D), jnp.float32),
  )(p0, p1, w)


def kernel(features, edge_index, edge_values, W):
  src = edge_index[0]
  dst = edge_index[1]
  partials = _sc_segment_sum(features, src, dst, edge_values)
  return _tc_combine_matmul(partials[0], partials[1], W)
